# Initial kernel scaffold; baseline (speedup 1.0000x reference)
import jax
import jax.numpy as jnp
from jax.experimental import pallas as pl


def kernel(input_features, patterns, alpha, W_enc, b_enc, W_dec, b_dec):
    raise NotImplementedError("write your pallas kernel here")


# speedup vs baseline: 25.7296x; 25.7296x over previous
import jax
import jax.numpy as jnp
from jax import lax
from jax.experimental import pallas as pl

TOPK = 128
_I32_MIN = jnp.iinfo(jnp.int32).min
_I32_MAX = jnp.iinfo(jnp.int32).max


def _orderable_keys(s):
    i = lax.bitcast_convert_type(s, jnp.int32)
    return jnp.where(i >= 0, i, i ^ jnp.int32(0x7FFFFFFF))


def _retrieve_kernel(x_ref, patT_ref, pat_ref, wenc_ref, benc_ref,
                     wdec_ref, bdec_ref, alpha_ref, out_ref):
    f32 = jnp.float32
    bf16 = jnp.bfloat16
    x = x_ref[...]
    enc = jnp.dot(x.astype(bf16), wenc_ref[...].astype(bf16),
                  preferred_element_type=f32) + benc_ref[...]
    en = enc / jnp.maximum(
        jnp.sqrt(jnp.sum(enc * enc, axis=1, keepdims=True)), 1e-8)

    patT = patT_ref[...]
    pn2 = jnp.sum(patT * patT, axis=0, keepdims=True)
    inv_pn = 1.0 / jnp.maximum(jnp.sqrt(pn2), 1e-8)
    pn_b = (patT * inv_pn).astype(bf16)
    sims = jnp.dot(en.astype(bf16), pn_b,
                   preferred_element_type=f32)

    keys = _orderable_keys(sims)
    br, p = keys.shape
    k = TOPK

    def body(_, carry):
        lo, hi, c_lo, c_hi = carry
        mid = (lo >> 1) + (hi >> 1) + (lo & hi & 1)
        c = jnp.sum((keys > mid).astype(jnp.int32), axis=1, keepdims=True)
        ge = c >= k
        return (jnp.where(ge, mid, lo), jnp.where(ge, hi, mid),
                jnp.where(ge, c, c_lo), jnp.where(ge, c_hi, c))

    lo0 = jnp.full((br, 1), _I32_MIN, jnp.int32)
    hi0 = jnp.full((br, 1), _I32_MAX, jnp.int32)
    c_lo0 = jnp.full((br, 1), p, jnp.int32)
    c_hi0 = jnp.zeros((br, 1), jnp.int32)
    lo, hi, c_lo, c_hi = lax.fori_loop(0, 32, body, (lo0, hi0, c_lo0, c_hi0))

    frac = (k - c_hi).astype(f32) / jnp.maximum(c_lo - c_hi, 1).astype(f32)
    w = jnp.where(keys > hi, f32(1.0), jnp.where(keys == hi, frac, f32(0.0)))

    pat_b = pat_ref[...].astype(bf16).astype(f32)
    avg = jnp.dot(w, pat_b, preferred_element_type=f32,
                  precision=lax.Precision.HIGHEST)
    wdec_b = wdec_ref[...].astype(bf16).astype(f32)
    dec = jnp.dot(avg, wdec_b, preferred_element_type=f32,
                  precision=lax.Precision.HIGHEST) * (1.0 / k) + bdec_ref[...]
    out_ref[...] = x + alpha_ref[0, 0] * dec


def kernel(input_features, patterns, alpha, W_enc, b_enc, W_dec, b_dec):
    b, l, d = input_features.shape
    p, pd = patterns.shape
    n = b * l
    x = input_features.reshape(n, d)
    pat_t = patterns.T

    br = 128
    out = pl.pallas_call(
        _retrieve_kernel,
        grid=(n // br,),
        in_specs=[
            pl.BlockSpec((br, d), lambda i: (i, 0)),
            pl.BlockSpec((pd, p), lambda i: (0, 0)),
            pl.BlockSpec((p, pd), lambda i: (0, 0)),
            pl.BlockSpec((d, pd), lambda i: (0, 0)),
            pl.BlockSpec((1, pd), lambda i: (0, 0)),
            pl.BlockSpec((pd, d), lambda i: (0, 0)),
            pl.BlockSpec((1, d), lambda i: (0, 0)),
            pl.BlockSpec((1, 1), lambda i: (0, 0)),
        ],
        out_specs=pl.BlockSpec((br, d), lambda i: (i, 0)),
        out_shape=jax.ShapeDtypeStruct((n, d), jnp.float32),
    )(x, pat_t, patterns, W_enc, b_enc.reshape(1, pd), W_dec,
      b_dec.reshape(1, d), alpha.reshape(1, 1))
    return out.reshape(b, l, d)


# speedup vs baseline: 37.8131x; 1.4696x over previous
import jax
import jax.numpy as jnp
from jax import lax
from jax.experimental import pallas as pl

TOPK = 128
_I32_MIN = jnp.iinfo(jnp.int32).min
_I32_MAX = jnp.iinfo(jnp.int32).max


def _orderable_keys(s):
    i = lax.bitcast_convert_type(s, jnp.int32)
    return jnp.where(i >= 0, i, i ^ jnp.int32(0x7FFFFFFF))


def _retrieve_kernel(x_ref, patT_ref, pat_ref, wenc_ref, benc_ref,
                     wdec_ref, bdec_ref, alpha_ref, out_ref):
    f32 = jnp.float32
    bf16 = jnp.bfloat16
    x = x_ref[...]
    enc = jnp.dot(x.astype(bf16), wenc_ref[...].astype(bf16),
                  preferred_element_type=f32) + benc_ref[...]
    en = enc / jnp.maximum(
        jnp.sqrt(jnp.sum(enc * enc, axis=1, keepdims=True)), 1e-8)

    patT = patT_ref[...]
    pn2 = jnp.sum(patT * patT, axis=0, keepdims=True)
    inv_pn = 1.0 / jnp.maximum(jnp.sqrt(pn2), 1e-8)
    pn_b = (patT * inv_pn).astype(bf16)
    sims = jnp.dot(en.astype(bf16), pn_b,
                   preferred_element_type=f32)

    keys = _orderable_keys(sims)
    br, p = keys.shape
    k = TOPK

    keys_hi = (keys >> 16).astype(jnp.int16)

    def count_i16(mask):
        a = mask.astype(jnp.int16)
        n = a.shape[1]
        while n > 256:
            n //= 2
            a = a[:, :n] + a[:, n:]
        return jnp.sum(a.astype(jnp.int32), axis=1, keepdims=True)

    def body1(_, carry):
        lo, hi, c_lo, c_hi = carry
        mid = (lo + hi) >> 1
        c = count_i16(keys_hi > mid.astype(jnp.int16))
        ge = c >= k
        return (jnp.where(ge, mid, lo), jnp.where(ge, hi, mid),
                jnp.where(ge, c, c_lo), jnp.where(ge, c_hi, c))

    lo1 = jnp.full((br, 1), -32768, jnp.int32)
    hi1 = jnp.full((br, 1), 32767, jnp.int32)
    c_lo1 = jnp.full((br, 1), p, jnp.int32)
    c_hi1 = jnp.zeros((br, 1), jnp.int32)
    lo1, hi1, c_lo1, c_hi1 = lax.fori_loop(
        0, 16, body1, (lo1, hi1, c_lo1, c_hi1))

    base = hi1 << 16

    def body2(_, carry):
        lo, hi, c_lo, c_hi = carry
        mid = lo + ((hi - lo) >> 1)
        c = jnp.sum((keys > mid).astype(jnp.int32), axis=1, keepdims=True)
        ge = c >= k
        return (jnp.where(ge, mid, lo), jnp.where(ge, hi, mid),
                jnp.where(ge, c, c_lo), jnp.where(ge, c_hi, c))

    lo2, hi2, c_lo2, c_hi2 = lax.fori_loop(
        0, 8, body2, (base - 1, base + 65535, c_lo1, c_hi1))

    frac = (k - c_hi2).astype(f32) / jnp.maximum(c_lo2 - c_hi2, 1).astype(f32)
    w = jnp.where(keys > hi2, f32(1.0), jnp.where(keys > lo2, frac, f32(0.0)))

    pat_b = pat_ref[...].astype(bf16).astype(f32)
    avg = jnp.dot(w, pat_b, preferred_element_type=f32,
                  precision=lax.Precision.HIGHEST)
    wdec_b = wdec_ref[...].astype(bf16).astype(f32)
    dec = jnp.dot(avg, wdec_b, preferred_element_type=f32,
                  precision=lax.Precision.HIGHEST) * (1.0 / k) + bdec_ref[...]
    out_ref[...] = x + alpha_ref[0, 0] * dec


def kernel(input_features, patterns, alpha, W_enc, b_enc, W_dec, b_dec):
    b, l, d = input_features.shape
    p, pd = patterns.shape
    n = b * l
    x = input_features.reshape(n, d)
    pat_t = patterns.T

    br = 128
    out = pl.pallas_call(
        _retrieve_kernel,
        grid=(n // br,),
        in_specs=[
            pl.BlockSpec((br, d), lambda i: (i, 0)),
            pl.BlockSpec((pd, p), lambda i: (0, 0)),
            pl.BlockSpec((p, pd), lambda i: (0, 0)),
            pl.BlockSpec((d, pd), lambda i: (0, 0)),
            pl.BlockSpec((1, pd), lambda i: (0, 0)),
            pl.BlockSpec((pd, d), lambda i: (0, 0)),
            pl.BlockSpec((1, d), lambda i: (0, 0)),
            pl.BlockSpec((1, 1), lambda i: (0, 0)),
        ],
        out_specs=pl.BlockSpec((br, d), lambda i: (i, 0)),
        out_shape=jax.ShapeDtypeStruct((n, d), jnp.float32),
    )(x, pat_t, patterns, W_enc, b_enc.reshape(1, pd), W_dec,
      b_dec.reshape(1, d), alpha.reshape(1, 1))
    return out.reshape(b, l, d)


# speedup vs baseline: 48.6838x; 1.2875x over previous
import jax
import jax.numpy as jnp
from jax import lax
from jax.experimental import pallas as pl

TOPK = 128


def _retrieve_kernel(x_ref, patT_ref, pat_ref, wenc_ref, benc_ref,
                     wdec_ref, bdec_ref, alpha_ref, out_ref):
    f32 = jnp.float32
    bf16 = jnp.bfloat16
    i16 = jnp.int16
    i32 = jnp.int32
    k = TOPK

    x = x_ref[...]
    enc = jnp.dot(x.astype(bf16), wenc_ref[...].astype(bf16),
                  preferred_element_type=f32) + benc_ref[...]
    en = enc / jnp.maximum(
        jnp.sqrt(jnp.sum(enc * enc, axis=1, keepdims=True)), 1e-8)

    patT = patT_ref[...]
    pn2 = jnp.sum(patT * patT, axis=0, keepdims=True)
    inv_pn = 1.0 / jnp.maximum(jnp.sqrt(pn2), 1e-8)
    pn_b = (patT * inv_pn).astype(bf16)
    sims = jnp.dot(en.astype(bf16), pn_b,
                   preferred_element_type=f32)

    ib = lax.bitcast_convert_type(sims, i32)
    keys = jnp.where(ib >= 0, ib, ib ^ i32(0x7FFFFFFF))
    br, p = keys.shape

    def count_i16(vals16):
        a = vals16
        n = a.shape[1]
        while n > 256:
            n //= 2
            a = a[:, :n] + a[:, n:]
        return jnp.sum(a.astype(i32), axis=1, keepdims=True)

    keys_hi = (keys >> 16).astype(i16)

    def body1(_, carry):
        lo, hi, c_lo, c_hi = carry
        mid = (lo + hi) >> 1
        c = count_i16((keys_hi > mid.astype(i16)).astype(i16))
        ge = c >= k
        return (jnp.where(ge, mid, lo), jnp.where(ge, hi, mid),
                jnp.where(ge, c, c_lo), jnp.where(ge, c_hi, c))

    lo1 = jnp.full((br, 1), -32768, i32)
    hi1 = jnp.full((br, 1), 32767, i32)
    lo1, hi1, c_lo1, c_hi1 = lax.fori_loop(
        0, 16, body1,
        (lo1, hi1, jnp.full((br, 1), p, i32), jnp.zeros((br, 1), i32)))

    in_band = keys_hi == hi1.astype(i16)
    lo_s16 = ((keys & i32(0xFFFF)) - 32768).astype(i16)
    lo_cmp = jnp.where(in_band, lo_s16, i16(-32768))

    def body2(_, carry):
        lo, hi, c_lo, c_hi = carry
        mid = lo + ((hi - lo) >> 1)
        c = c_hi1 + count_i16((lo_cmp > mid.astype(i16)).astype(i16))
        ge = c >= k
        return (jnp.where(ge, mid, lo), jnp.where(ge, hi, mid),
                jnp.where(ge, c, c_lo), jnp.where(ge, c_hi, c))

    lo2, hi2, c_lo2, c_hi2 = lax.fori_loop(
        0, 8, body2,
        (jnp.full((br, 1), -32769, i32), jnp.full((br, 1), 32767, i32),
         c_lo1, c_hi1))

    base = hi1 << 16
    lo_f = base + (lo2 + 32768)
    hi_f = base + (hi2 + 32768)

    frac = (k - c_hi2).astype(f32) / jnp.maximum(c_lo2 - c_hi2, 1).astype(f32)
    w = jnp.where(keys > hi_f, f32(1.0),
                  jnp.where(keys > lo_f, frac, f32(0.0))).astype(bf16)

    pat_b = pat_ref[...].astype(bf16)
    avg = jnp.dot(w, pat_b, preferred_element_type=f32)
    wdec_b = wdec_ref[...].astype(bf16).astype(f32)
    dec = jnp.dot(avg, wdec_b, preferred_element_type=f32,
                  precision=lax.Precision.HIGHEST) * (1.0 / k) + bdec_ref[...]
    out_ref[...] = x + alpha_ref[0, 0] * dec


def kernel(input_features, patterns, alpha, W_enc, b_enc, W_dec, b_dec):
    b, l, d = input_features.shape
    p, pd = patterns.shape
    n = b * l
    x = input_features.reshape(n, d)
    pat_t = patterns.T

    br = 128
    out = pl.pallas_call(
        _retrieve_kernel,
        grid=(n // br,),
        in_specs=[
            pl.BlockSpec((br, d), lambda i: (i, 0)),
            pl.BlockSpec((pd, p), lambda i: (0, 0)),
            pl.BlockSpec((p, pd), lambda i: (0, 0)),
            pl.BlockSpec((d, pd), lambda i: (0, 0)),
            pl.BlockSpec((1, pd), lambda i: (0, 0)),
            pl.BlockSpec((pd, d), lambda i: (0, 0)),
            pl.BlockSpec((1, d), lambda i: (0, 0)),
            pl.BlockSpec((1, 1), lambda i: (0, 0)),
        ],
        out_specs=pl.BlockSpec((br, d), lambda i: (i, 0)),
        out_shape=jax.ShapeDtypeStruct((n, d), jnp.float32),
    )(x, pat_t, patterns, W_enc, b_enc.reshape(1, pd), W_dec,
      b_dec.reshape(1, d), alpha.reshape(1, 1))
    return out.reshape(b, l, d)


# speedup vs baseline: 51.7343x; 1.0627x over previous
import jax
import jax.numpy as jnp
from jax import lax
from jax.experimental import pallas as pl

TOPK = 128


def _normalize_kernel(patT_ref, pnb_ref):
    patT = patT_ref[...]
    pn2 = jnp.sum(patT * patT, axis=0, keepdims=True)
    inv_pn = 1.0 / jnp.maximum(jnp.sqrt(pn2), 1e-8)
    pnb_ref[...] = (patT * inv_pn).astype(jnp.bfloat16)


def _retrieve_kernel(x_ref, pnb_ref, patb_ref, wenc_ref, benc_ref,
                     wdec_ref, bdec_ref, alpha_ref, out_ref):
    f32 = jnp.float32
    bf16 = jnp.bfloat16
    i16 = jnp.int16
    i32 = jnp.int32
    k = TOPK

    x = x_ref[...]
    enc = jnp.dot(x.astype(bf16), wenc_ref[...].astype(bf16),
                  preferred_element_type=f32) + benc_ref[...]
    en = enc / jnp.maximum(
        jnp.sqrt(jnp.sum(enc * enc, axis=1, keepdims=True)), 1e-8)

    sims = jnp.dot(en.astype(bf16), pnb_ref[...],
                   preferred_element_type=f32)

    ib = lax.bitcast_convert_type(sims, i32)
    keys = jnp.where(ib >= 0, ib, ib ^ i32(0x7FFFFFFF))
    br, p = keys.shape

    def count_i16(vals16):
        a = vals16
        n = a.shape[1]
        while n > 256:
            n //= 2
            a = a[:, :n] + a[:, n:]
        return jnp.sum(a.astype(i32), axis=1, keepdims=True)

    keys_hi = (keys >> 16).astype(i16)

    def body1(_, carry):
        lo, hi, c_lo, c_hi = carry
        mid = (lo + hi) >> 1
        c = count_i16((keys_hi > mid.astype(i16)).astype(i16))
        ge = c >= k
        return (jnp.where(ge, mid, lo), jnp.where(ge, hi, mid),
                jnp.where(ge, c, c_lo), jnp.where(ge, c_hi, c))

    lo1 = jnp.full((br, 1), -32768, i32)
    hi1 = jnp.full((br, 1), 32767, i32)
    lo1, hi1, c_lo1, c_hi1 = lax.fori_loop(
        0, 16, body1,
        (lo1, hi1, jnp.full((br, 1), p, i32), jnp.zeros((br, 1), i32)))

    in_band = keys_hi == hi1.astype(i16)
    lo_s16 = ((keys & i32(0xFFFF)) - 32768).astype(i16)
    lo_cmp = jnp.where(in_band, lo_s16, i16(-32768))

    def body2(_, carry):
        lo, hi, c_lo, c_hi = carry
        mid = lo + ((hi - lo) >> 1)
        c = c_hi1 + count_i16((lo_cmp > mid.astype(i16)).astype(i16))
        ge = c >= k
        return (jnp.where(ge, mid, lo), jnp.where(ge, hi, mid),
                jnp.where(ge, c, c_lo), jnp.where(ge, c_hi, c))

    lo2, hi2, c_lo2, c_hi2 = lax.fori_loop(
        0, 8, body2,
        (jnp.full((br, 1), -32769, i32), jnp.full((br, 1), 32767, i32),
         c_lo1, c_hi1))

    base = hi1 << 16
    lo_f = base + (lo2 + 32768)
    hi_f = base + (hi2 + 32768)

    frac = (k - c_hi2).astype(f32) / jnp.maximum(c_lo2 - c_hi2, 1).astype(f32)
    w = jnp.where(keys > hi_f, f32(1.0),
                  jnp.where(keys > lo_f, frac, f32(0.0))).astype(bf16)

    avg = jnp.dot(w, patb_ref[...],
                  preferred_element_type=f32)
    wdec_b = wdec_ref[...].astype(bf16).astype(f32)
    dec = jnp.dot(avg, wdec_b, preferred_element_type=f32,
                  precision=lax.Precision.HIGHEST) * (1.0 / k) + bdec_ref[...]
    out_ref[...] = x + alpha_ref[0, 0] * dec


def kernel(input_features, patterns, alpha, W_enc, b_enc, W_dec, b_dec):
    b, l, d = input_features.shape
    p, pd = patterns.shape
    n = b * l
    x = input_features.reshape(n, d)
    pat_t = patterns.T
    pat_b = patterns.astype(jnp.bfloat16)

    pn_b = pl.pallas_call(
        _normalize_kernel,
        out_shape=jax.ShapeDtypeStruct((pd, p), jnp.bfloat16),
    )(pat_t)

    br = 128
    out = pl.pallas_call(
        _retrieve_kernel,
        grid=(n // br,),
        in_specs=[
            pl.BlockSpec((br, d), lambda i: (i, 0)),
            pl.BlockSpec((pd, p), lambda i: (0, 0)),
            pl.BlockSpec((p, pd), lambda i: (0, 0)),
            pl.BlockSpec((d, pd), lambda i: (0, 0)),
            pl.BlockSpec((1, pd), lambda i: (0, 0)),
            pl.BlockSpec((pd, d), lambda i: (0, 0)),
            pl.BlockSpec((1, d), lambda i: (0, 0)),
            pl.BlockSpec((1, 1), lambda i: (0, 0)),
        ],
        out_specs=pl.BlockSpec((br, d), lambda i: (i, 0)),
        out_shape=jax.ShapeDtypeStruct((n, d), jnp.float32),
    )(x, pn_b, pat_b, W_enc, b_enc.reshape(1, pd), W_dec,
      b_dec.reshape(1, d), alpha.reshape(1, 1))
    return out.reshape(b, l, d)


# speedup vs baseline: 58.0775x; 1.1226x over previous
import jax
import jax.numpy as jnp
from jax import lax
from jax.experimental import pallas as pl

TOPK = 128


def _normalize_kernel(patT_ref, pnb_ref):
    patT = patT_ref[...]
    pn2 = jnp.sum(patT * patT, axis=0, keepdims=True)
    inv_pn = 1.0 / jnp.maximum(jnp.sqrt(pn2), 1e-8)
    pnb_ref[...] = (patT * inv_pn).astype(jnp.bfloat16)


def _retrieve_kernel(x_ref, pnb_ref, patb_ref, wenc_ref, benc_ref,
                     wdec_ref, bdec_ref, alpha_ref, out_ref):
    f32 = jnp.float32
    bf16 = jnp.bfloat16
    i16 = jnp.int16
    i32 = jnp.int32
    k = TOPK

    x = x_ref[...]
    enc = jnp.dot(x.astype(bf16), wenc_ref[...].astype(bf16),
                  preferred_element_type=f32) + benc_ref[...]
    en = enc / jnp.maximum(
        jnp.sqrt(jnp.sum(enc * enc, axis=1, keepdims=True)), 1e-8)

    sims = jnp.dot(en.astype(bf16), pnb_ref[...],
                   preferred_element_type=f32)

    ib = lax.bitcast_convert_type(sims, i32)
    keys = jnp.where(ib >= 0, ib, ib ^ i32(0x7FFFFFFF))
    br, p = keys.shape

    def count_i16(vals16):
        a = vals16
        n = a.shape[1]
        while n > 256:
            n //= 2
            a = a[:, :n] + a[:, n:]
        return jnp.sum(a.astype(i32), axis=1, keepdims=True)

    keys_hi = (keys >> 16).astype(i16)

    def body1(_, carry):
        lo, hi, c_lo, c_hi = carry
        mid = (lo + hi) >> 1
        c = count_i16((keys_hi > mid.astype(i16)).astype(i16))
        ge = c >= k
        return (jnp.where(ge, mid, lo), jnp.where(ge, hi, mid),
                jnp.where(ge, c, c_lo), jnp.where(ge, c_hi, c))

    lo1 = jnp.full((br, 1), -32768, i32)
    hi1 = jnp.full((br, 1), 32767, i32)
    lo1, hi1, c_lo1, c_hi1 = lax.fori_loop(
        0, 16, body1,
        (lo1, hi1, jnp.full((br, 1), p, i32), jnp.zeros((br, 1), i32)))

    in_band = keys_hi == hi1.astype(i16)
    lo_s16 = ((keys & i32(0xFFFF)) - 32768).astype(i16)
    lo_cmp = jnp.where(in_band, lo_s16, i16(-32768))

    def body2(_, carry):
        lo, hi, c_lo, c_hi = carry
        mid = lo + ((hi - lo) >> 1)
        c = c_hi1 + count_i16((lo_cmp > mid.astype(i16)).astype(i16))
        ge = c >= k
        return (jnp.where(ge, mid, lo), jnp.where(ge, hi, mid),
                jnp.where(ge, c, c_lo), jnp.where(ge, c_hi, c))

    lo2, hi2, c_lo2, c_hi2 = lax.fori_loop(
        0, 4, body2,
        (jnp.full((br, 1), -32769, i32), jnp.full((br, 1), 32767, i32),
         c_lo1, c_hi1))

    base = hi1 << 16
    lo_f = base + (lo2 + 32768)
    hi_f = base + (hi2 + 32768)

    frac = (k - c_hi2).astype(f32) / jnp.maximum(c_lo2 - c_hi2, 1).astype(f32)
    w = jnp.where(keys > hi_f, f32(1.0),
                  jnp.where(keys > lo_f, frac, f32(0.0))).astype(bf16)

    avg = jnp.dot(w, patb_ref[...],
                  preferred_element_type=f32)
    wdec_b = wdec_ref[...].astype(bf16).astype(f32)
    dec = jnp.dot(avg, wdec_b, preferred_element_type=f32,
                  precision=lax.Precision.HIGHEST) * (1.0 / k) + bdec_ref[...]
    out_ref[...] = x + alpha_ref[0, 0] * dec


def kernel(input_features, patterns, alpha, W_enc, b_enc, W_dec, b_dec):
    b, l, d = input_features.shape
    p, pd = patterns.shape
    n = b * l
    x = input_features.reshape(n, d)
    pat_t = patterns.T
    pat_b = patterns.astype(jnp.bfloat16)

    pn_b = pl.pallas_call(
        _normalize_kernel,
        out_shape=jax.ShapeDtypeStruct((pd, p), jnp.bfloat16),
    )(pat_t)

    br = 128
    out = pl.pallas_call(
        _retrieve_kernel,
        grid=(n // br,),
        in_specs=[
            pl.BlockSpec((br, d), lambda i: (i, 0)),
            pl.BlockSpec((pd, p), lambda i: (0, 0)),
            pl.BlockSpec((p, pd), lambda i: (0, 0)),
            pl.BlockSpec((d, pd), lambda i: (0, 0)),
            pl.BlockSpec((1, pd), lambda i: (0, 0)),
            pl.BlockSpec((pd, d), lambda i: (0, 0)),
            pl.BlockSpec((1, d), lambda i: (0, 0)),
            pl.BlockSpec((1, 1), lambda i: (0, 0)),
        ],
        out_specs=pl.BlockSpec((br, d), lambda i: (i, 0)),
        out_shape=jax.ShapeDtypeStruct((n, d), jnp.float32),
    )(x, pn_b, pat_b, W_enc, b_enc.reshape(1, pd), W_dec,
      b_dec.reshape(1, d), alpha.reshape(1, 1))
    return out.reshape(b, l, d)


# speedup vs baseline: 60.0922x; 1.0347x over previous
import jax
import jax.numpy as jnp
from jax import lax
from jax.experimental import pallas as pl

TOPK = 128


def _normalize_kernel(patT_ref, pnb_ref):
    patT = patT_ref[...]
    pn2 = jnp.sum(patT * patT, axis=0, keepdims=True)
    inv_pn = 1.0 / jnp.maximum(jnp.sqrt(pn2), 1e-8)
    pnb_ref[...] = (patT * inv_pn).astype(jnp.bfloat16)


def _retrieve_kernel(x_ref, pnb_ref, patb_ref, wenc_ref, benc_ref,
                     wdec_ref, bdec_ref, alpha_ref, out_ref):
    f32 = jnp.float32
    bf16 = jnp.bfloat16
    i16 = jnp.int16
    i32 = jnp.int32
    k = TOPK

    x = x_ref[...]
    enc = jnp.dot(x.astype(bf16), wenc_ref[...].astype(bf16),
                  preferred_element_type=f32) + benc_ref[...]
    en = enc / jnp.maximum(
        jnp.sqrt(jnp.sum(enc * enc, axis=1, keepdims=True)), 1e-8)

    sims = jnp.dot(en.astype(bf16), pnb_ref[...],
                   preferred_element_type=f32)

    ib = lax.bitcast_convert_type(sims, i32)
    keys = jnp.where(ib >= 0, ib, ib ^ i32(0x7FFFFFFF))
    br, p = keys.shape

    def count_i16(vals16):
        a = vals16
        n = a.shape[1]
        while n > 256:
            n //= 2
            a = a[:, :n] + a[:, n:]
        return jnp.sum(a.astype(i32), axis=1, keepdims=True)

    keys_hi = (keys >> 16).astype(i16)

    def body1(_, carry):
        lo, hi, c_lo, c_hi = carry
        mid = (lo + hi) >> 1
        c = count_i16((keys_hi > mid.astype(i16)).astype(i16))
        ge = c >= k
        return (jnp.where(ge, mid, lo), jnp.where(ge, hi, mid),
                jnp.where(ge, c, c_lo), jnp.where(ge, c_hi, c))

    lo1 = jnp.full((br, 1), -16384, i32)
    hi1 = jnp.full((br, 1), 16383, i32)
    lo1, hi1, c_lo1, c_hi1 = lax.fori_loop(
        0, 15, body1,
        (lo1, hi1, jnp.full((br, 1), p, i32), jnp.zeros((br, 1), i32)))

    in_band = keys_hi == hi1.astype(i16)
    lo_s16 = ((keys & i32(0xFFFF)) - 32768).astype(i16)
    lo_cmp = jnp.where(in_band, lo_s16, i16(-32768))

    def body2(_, carry):
        lo, hi, c_lo, c_hi = carry
        mid = lo + ((hi - lo) >> 1)
        c = c_hi1 + count_i16((lo_cmp > mid.astype(i16)).astype(i16))
        ge = c >= k
        return (jnp.where(ge, mid, lo), jnp.where(ge, hi, mid),
                jnp.where(ge, c, c_lo), jnp.where(ge, c_hi, c))

    lo2, hi2, c_lo2, c_hi2 = lax.fori_loop(
        0, 4, body2,
        (jnp.full((br, 1), -32769, i32), jnp.full((br, 1), 32767, i32),
         c_lo1, c_hi1))

    base = hi1 << 16
    lo_f = base + (lo2 + 32768)
    hi_f = base + (hi2 + 32768)

    frac = (k - c_hi2).astype(f32) / jnp.maximum(c_lo2 - c_hi2, 1).astype(f32)
    w = jnp.where(keys > hi_f, f32(1.0),
                  jnp.where(keys > lo_f, frac, f32(0.0))).astype(bf16)

    avg = jnp.dot(w, patb_ref[...],
                  preferred_element_type=f32)
    wdec_b = wdec_ref[...].astype(bf16).astype(f32)
    dec = jnp.dot(avg, wdec_b, preferred_element_type=f32,
                  precision=lax.Precision.HIGHEST) * (1.0 / k) + bdec_ref[...]
    out_ref[...] = x + alpha_ref[0, 0] * dec


def kernel(input_features, patterns, alpha, W_enc, b_enc, W_dec, b_dec):
    b, l, d = input_features.shape
    p, pd = patterns.shape
    n = b * l
    x = input_features.reshape(n, d)
    pat_t = patterns.T
    pat_b = patterns.astype(jnp.bfloat16)

    pn_b = pl.pallas_call(
        _normalize_kernel,
        out_shape=jax.ShapeDtypeStruct((pd, p), jnp.bfloat16),
    )(pat_t)

    br = 128
    out = pl.pallas_call(
        _retrieve_kernel,
        grid=(n // br,),
        in_specs=[
            pl.BlockSpec((br, d), lambda i: (i, 0)),
            pl.BlockSpec((pd, p), lambda i: (0, 0)),
            pl.BlockSpec((p, pd), lambda i: (0, 0)),
            pl.BlockSpec((d, pd), lambda i: (0, 0)),
            pl.BlockSpec((1, pd), lambda i: (0, 0)),
            pl.BlockSpec((pd, d), lambda i: (0, 0)),
            pl.BlockSpec((1, d), lambda i: (0, 0)),
            pl.BlockSpec((1, 1), lambda i: (0, 0)),
        ],
        out_specs=pl.BlockSpec((br, d), lambda i: (i, 0)),
        out_shape=jax.ShapeDtypeStruct((n, d), jnp.float32),
    )(x, pn_b, pat_b, W_enc, b_enc.reshape(1, pd), W_dec,
      b_dec.reshape(1, d), alpha.reshape(1, 1))
    return out.reshape(b, l, d)
